# baseline (speedup 1.0000x reference)
import functools
import jax
import jax.numpy as jnp
from jax import lax
from jax.experimental import pallas as pl
from jax.experimental.pallas import tpu as pltpu
from jax.experimental.pallas import tpu_sc as plsc

_C = 26
_D = 100000
_F = 32
_B = 4096
_NW = 32
_BPW = _B // _NW
_RB = _BPW // 16
_VR = (_C * _D * _F) // 128


def _sc_body(cond_t, table, out, t_v, idx0_v, idx1_v, w_v, o0_v, v0_b, v1_b,
             out_v, sem0, sem1):
    cid = lax.axis_index("c")
    sid = lax.axis_index("s")
    wid = sid * 2 + cid
    base = wid * _BPW

    def dim_body(i, _):
        pltpu.sync_copy(cond_t.at[i, pl.ds(base, _BPW)], t_v)
        for j in range(_RB):
            t = t_v[pl.ds(j * 16, 16)]
            ts = t * float(_D - 1)
            i0 = ts.astype(jnp.int32)
            w = ts - i0.astype(jnp.float32)
            g0 = i0 + i * _D
            o0 = jnp.bitwise_and(g0, 3)
            idx0_v[pl.ds(j * 16, 16)] = lax.shift_right_logical(g0, 2)
            idx1_v[pl.ds(j * 16, 16)] = lax.shift_right_logical(g0 + 1, 2)
            o0_v[pl.ds(j * 16, 16)] = o0
            w_v[pl.ds(j * 16, 16)] = w
        cp0 = pltpu.async_copy(table.at[idx0_v], v0_b, sem0)
        cp1 = pltpu.async_copy(table.at[idx1_v], v1_b, sem1)
        cp0.wait()
        cp1.wait()

        def blk_body(rb, _):
            wv16 = w_v[pl.ds(rb * 16, 16)]
            ov16 = o0_v[pl.ds(rb * 16, 16)]
            base_r = rb * 16
            for l in range(16):
                wv = jnp.full((16,), wv16[l], jnp.float32)
                c0 = ov16[l] * _F
                c1 = jnp.bitwise_and(ov16[l] + 1, 3) * _F
                r = base_r + l
                for h in range(_F // 16):
                    a = v0_b[r, pl.ds(c0 + h * 16, 16)]
                    b = v1_b[r, pl.ds(c1 + h * 16, 16)]
                    out_v[r, pl.ds(h * 16, 16)] = a + wv * (b - a)
            return 0

        lax.fori_loop(0, _RB, blk_body, 0)
        pltpu.sync_copy(out_v, out.at[i, pl.ds(base, _BPW), :])
        return 0

    lax.fori_loop(0, _C, dim_body, 0)


_sc_kernel = functools.partial(
    pl.kernel,
    out_type=jax.ShapeDtypeStruct((_C, _B, _F), jnp.float32),
    mesh=plsc.VectorSubcoreMesh(core_axis_name="c", subcore_axis_name="s"),
    scratch_types=[
        pltpu.VMEM((_BPW,), jnp.float32),
        pltpu.VMEM((_BPW,), jnp.int32),
        pltpu.VMEM((_BPW,), jnp.int32),
        pltpu.VMEM((_BPW,), jnp.float32),
        pltpu.VMEM((_BPW,), jnp.int32),
        pltpu.VMEM((_BPW, 128), jnp.float32),
        pltpu.VMEM((_BPW, 128), jnp.float32),
        pltpu.VMEM((_BPW, _F), jnp.float32),
        pltpu.SemaphoreType.DMA,
        pltpu.SemaphoreType.DMA,
    ],
)(_sc_body)


@jax.jit
def kernel(cond, lines):
    cond_t = cond.T
    table = lines.reshape(_VR, 128)
    out3 = _sc_kernel(cond_t, table)
    return out3.transpose(1, 0, 2).reshape(_B, _C * _F)


# speedup vs baseline: 1.0627x; 1.0627x over previous
import functools
import jax
import jax.numpy as jnp
from jax import lax
from jax.experimental import pallas as pl
from jax.experimental.pallas import tpu as pltpu
from jax.experimental.pallas import tpu_sc as plsc

_C = 26
_D = 100000
_F = 32
_B = 4096
_NW = 32
_BPW = _B // _NW
_RB = _BPW // 16


def _sc_body(cond_t, table, out, t_v, idx0_v, idx1_v, w_v, v0_b, v1_b,
             out_v, sem0, sem1):
    cid = lax.axis_index("c")
    sid = lax.axis_index("s")
    wid = sid * 2 + cid
    base = wid * _BPW

    def dim_body(i, _):
        pltpu.sync_copy(cond_t.at[i, pl.ds(base, _BPW)], t_v)
        for j in range(_RB):
            t = t_v[pl.ds(j * 16, 16)]
            ts = t * float(_D - 1)
            i0 = ts.astype(jnp.int32)
            w = ts - i0.astype(jnp.float32)
            idx0_v[pl.ds(j * 16, 16)] = i0
            idx1_v[pl.ds(j * 16, 16)] = i0 + 1
            w_v[pl.ds(j * 16, 16)] = w
        cp0 = pltpu.async_copy(table.at[i].at[idx0_v], v0_b, sem0)
        cp1 = pltpu.async_copy(table.at[i].at[idx1_v], v1_b, sem1)
        cp0.wait()
        cp1.wait()

        def blk_body(rb, _):
            wv16 = w_v[pl.ds(rb * 16, 16)]
            base_r = rb * 16
            for l in range(16):
                wv = jnp.full((16,), wv16[l], jnp.float32)
                r = base_r + l
                for h in range(_F // 16):
                    a = v0_b[r, pl.ds(h * 16, 16)]
                    b = v1_b[r, pl.ds(h * 16, 16)]
                    out_v[r, pl.ds(h * 16, 16)] = a + wv * (b - a)
            return 0

        lax.fori_loop(0, _RB, blk_body, 0)
        pltpu.sync_copy(out_v, out.at[pl.ds(base, _BPW), pl.ds(i * _F, _F)])
        return 0

    lax.fori_loop(0, _C, dim_body, 0)


_sc_kernel = functools.partial(
    pl.kernel,
    out_type=jax.ShapeDtypeStruct((_B, _C * _F), jnp.float32),
    mesh=plsc.VectorSubcoreMesh(core_axis_name="c", subcore_axis_name="s"),
    compiler_params=pltpu.CompilerParams(use_tc_tiling_on_sc=False),
    scratch_types=[
        pltpu.VMEM((_BPW,), jnp.float32),
        pltpu.VMEM((_BPW,), jnp.int32),
        pltpu.VMEM((_BPW,), jnp.int32),
        pltpu.VMEM((_BPW,), jnp.float32),
        pltpu.VMEM((_BPW, _F), jnp.float32),
        pltpu.VMEM((_BPW, _F), jnp.float32),
        pltpu.VMEM((_BPW, _F), jnp.float32),
        pltpu.SemaphoreType.DMA,
        pltpu.SemaphoreType.DMA,
    ],
)(_sc_body)


@jax.jit
def kernel(cond, lines):
    cond_t = cond.T
    return _sc_kernel(cond_t, lines)
